# baseline (speedup 1.0000x reference)
import functools

import jax
import jax.numpy as jnp
from jax import lax
from jax.experimental import pallas as pl
from jax.experimental.pallas import tpu as pltpu
from jax.experimental.pallas import tpu_sc as plsc

NC = 2
NS = 16
LANES = 16
W = 33


def _tc_transform(feats, Wl, Wr):
  N, D = feats.shape
  H = Wl.shape[1]
  RB = 1024
  grid = (N // RB,)

  def body(f_ref, wl_ref, wr_ref, xlp_ref, xr_ref):
    f = f_ref[...]
    xl = jnp.dot(f, wl_ref[...], preferred_element_type=jnp.float32)
    xr = jnp.dot(f, wr_ref[...], preferred_element_type=jnp.float32)
    ones = jnp.ones((RB, 1), jnp.float32)
    zeros = jnp.zeros((RB, 1), jnp.float32)
    xlp_ref[...] = jnp.concatenate([xl, ones], axis=1)
    xr_ref[...] = jnp.concatenate([xr, zeros], axis=1)

  return pl.pallas_call(
      body,
      grid=grid,
      in_specs=[
          pl.BlockSpec((RB, D), lambda i: (i, 0)),
          pl.BlockSpec((D, H), lambda i: (0, 0)),
          pl.BlockSpec((D, H), lambda i: (0, 0)),
      ],
      out_specs=[
          pl.BlockSpec((RB, W), lambda i: (i, 0)),
          pl.BlockSpec((RB, W), lambda i: (i, 0)),
      ],
      out_shape=[
          jax.ShapeDtypeStruct((N, W), jnp.float32),
          jax.ShapeDtypeStruct((N, W), jnp.float32),
      ],
  )(feats, Wl, Wr)


def _tc_combine_transform(part, b, Wl, Wr):
  _, N, _ = part.shape
  H = Wl.shape[1]
  RB = 1024
  grid = (N // RB,)

  def body(p_ref, b_ref, wl_ref, wr_ref, xlp_ref, xr_ref):
    p = p_ref[0] + p_ref[1]
    den = jnp.maximum(p[:, H:H + 1], 1e-16)
    h = jnp.maximum(p[:, :H] / den + b_ref[...], 0.0)
    xl = jnp.dot(h, wl_ref[...], preferred_element_type=jnp.float32)
    xr = jnp.dot(h, wr_ref[...], preferred_element_type=jnp.float32)
    ones = jnp.ones((RB, 1), jnp.float32)
    zeros = jnp.zeros((RB, 1), jnp.float32)
    xlp_ref[...] = jnp.concatenate([xl, ones], axis=1)
    xr_ref[...] = jnp.concatenate([xr, zeros], axis=1)

  return pl.pallas_call(
      body,
      grid=grid,
      in_specs=[
          pl.BlockSpec((NC, RB, W), lambda i: (0, i, 0)),
          pl.BlockSpec((1, H), lambda i: (0, 0)),
          pl.BlockSpec((H, H), lambda i: (0, 0)),
          pl.BlockSpec((H, H), lambda i: (0, 0)),
      ],
      out_specs=[
          pl.BlockSpec((RB, W), lambda i: (i, 0)),
          pl.BlockSpec((RB, W), lambda i: (i, 0)),
      ],
      out_shape=[
          jax.ShapeDtypeStruct((N, W), jnp.float32),
          jax.ShapeDtypeStruct((N, W), jnp.float32),
      ],
  )(part, b, Wl, Wr)


def _tc_finalize(part, b):
  _, N, _ = part.shape
  H = W - 1
  RB = 1024
  grid = (N // RB,)

  def body(p_ref, b_ref, out_ref):
    p = p_ref[0] + p_ref[1]
    den = jnp.maximum(p[:, H:H + 1], 1e-16)
    out_ref[...] = p[:, :H] / den + b_ref[...]

  return pl.pallas_call(
      body,
      grid=grid,
      in_specs=[
          pl.BlockSpec((NC, RB, W), lambda i: (0, i, 0)),
          pl.BlockSpec((1, H), lambda i: (0, 0)),
      ],
      out_specs=pl.BlockSpec((RB, H), lambda i: (i, 0)),
      out_shape=jax.ShapeDtypeStruct((N, H), jnp.float32),
  )(part, b)


def _sc_edge_pass(xlp, xr, src2, dst2, atab, zeros, *, N, E, H, C):
  G = C // LANES
  per_worker = E // (NC * NS)
  nchunks = per_worker // C
  npairs = (nchunks - 1) // 2
  rows_per = N // NS

  mesh = plsc.VectorSubcoreMesh(core_axis_name="c", subcore_axis_name="s")

  @functools.partial(
      pl.kernel,
      out_type=jax.ShapeDtypeStruct((NC, N, W), jnp.float32),
      mesh=mesh,
      compiler_params=pltpu.CompilerParams(
          use_tc_tiling_on_sc=False, needs_layout_passes=False),
      scratch_types=[
          pltpu.VMEM((nchunks, C), jnp.int32),
          pltpu.VMEM((nchunks, C), jnp.int32),
          pltpu.VMEM((C, W), jnp.float32),
          pltpu.VMEM((C, W), jnp.float32),
          pltpu.VMEM((C, W), jnp.float32),
          pltpu.VMEM((C, W), jnp.float32),
          pltpu.VMEM((C, W), jnp.float32),
          pltpu.VMEM((C, W), jnp.float32),
          pltpu.VMEM((H, LANES), jnp.float32),
          pltpu.VMEM_SHARED((N, W), jnp.float32),
          pltpu.SemaphoreType.DMA,
          pltpu.SemaphoreType.DMA,
          pltpu.SemaphoreType.DMA,
          pltpu.SemaphoreType.DMA,
          pltpu.SemaphoreType.DMA,
          pltpu.SemaphoreType.DMA,
      ],
  )
  def sc_kernel(xlp_hbm, xr_hbm, src_hbm, dst_hbm, atab_hbm, zeros_hbm,
                out_hbm, src_a, dst_a, xlp_b0, xr_b0, xlp_b1, xr_b1, w_b0,
                w_b1, atab_v, num_sh, sl0, sr0, sl1, sr1, sw0, sw1):
    cid = lax.axis_index("c")
    sid = lax.axis_index("s")
    wid = sid * NC + cid

    pltpu.sync_copy(src_hbm.at[pl.ds(wid * nchunks, nchunks)], src_a)
    pltpu.sync_copy(dst_hbm.at[pl.ds(wid * nchunks, nchunks)], dst_a)
    pltpu.sync_copy(atab_hbm, atab_v)
    pltpu.async_copy(xlp_hbm.at[src_a.at[0]], xlp_b0, sl0)
    pltpu.async_copy(xr_hbm.at[dst_a.at[0]], xr_b0, sr0)
    pltpu.sync_copy(zeros_hbm.at[pl.ds(sid * rows_per, rows_per)],
                    num_sh.at[pl.ds(sid * rows_per, rows_per)])
    plsc.subcore_barrier()

    iota = lax.iota(jnp.int32, LANES)
    rows_g = [iota + g * LANES for g in range(G)]
    col_den = jnp.full((LANES,), H, jnp.int32)

    def compute_chunk(xlp_b, xr_b, w_b, sw, j, wait_w):
      hvs = [jnp.full((LANES,), h, jnp.int32) for h in range(H)]
      accs = [jnp.zeros((LANES,), jnp.float32) for _ in range(G)]
      for h in range(H):
        a_h = plsc.load_gather(atab_v, [hvs[h], iota])
        for g in range(G):
          tl = plsc.load_gather(xlp_b, [rows_g[g], hvs[h]])
          tr = plsc.load_gather(xr_b, [rows_g[g], hvs[h]])
          t = tl + tr
          t = jnp.maximum(t, 0.2 * t)
          accs[g] = accs[g] + a_h * t
      ees = [jnp.exp(jnp.minimum(a, 60.0)) for a in accs]

      @pl.when(wait_w)
      def _():
        pltpu.make_async_copy(w_b, num_sh.at[dst_a.at[j]], sw).wait()

      for g in range(G):
        plsc.store_scatter(w_b, [rows_g[g], col_den], ees[g])
      for h in range(H):
        for g in range(G):
          tl = plsc.load_gather(xlp_b, [rows_g[g], hvs[h]])
          plsc.store_scatter(w_b, [rows_g[g], hvs[h]], ees[g] * tl)

      pltpu.async_copy(w_b, num_sh.at[dst_a.at[j]], sw, add=True)

    def pair_body(i, carry):
      j0 = 2 * i
      pltpu.make_async_copy(xlp_hbm.at[src_a.at[j0]], xlp_b0, sl0).wait()
      pltpu.make_async_copy(xr_hbm.at[dst_a.at[j0]], xr_b0, sr0).wait()
      pltpu.async_copy(xlp_hbm.at[src_a.at[j0 + 1]], xlp_b1, sl1)
      pltpu.async_copy(xr_hbm.at[dst_a.at[j0 + 1]], xr_b1, sr1)
      compute_chunk(xlp_b0, xr_b0, w_b0, sw0, j0, i > 0)
      pltpu.make_async_copy(xlp_hbm.at[src_a.at[j0 + 1]], xlp_b1, sl1).wait()
      pltpu.make_async_copy(xr_hbm.at[dst_a.at[j0 + 1]], xr_b1, sr1).wait()
      pltpu.async_copy(xlp_hbm.at[src_a.at[j0 + 2]], xlp_b0, sl0)
      pltpu.async_copy(xr_hbm.at[dst_a.at[j0 + 2]], xr_b0, sr0)
      compute_chunk(xlp_b1, xr_b1, w_b1, sw1, j0 + 1, i > 0)
      return carry

    lax.fori_loop(0, npairs, pair_body, 0)

    last = nchunks - 1
    pltpu.make_async_copy(xlp_hbm.at[src_a.at[last]], xlp_b0, sl0).wait()
    pltpu.make_async_copy(xr_hbm.at[dst_a.at[last]], xr_b0, sr0).wait()
    compute_chunk(xlp_b0, xr_b0, w_b0, sw0, last, jnp.bool_(True))
    pltpu.make_async_copy(w_b0, num_sh.at[dst_a.at[last]], sw0).wait()
    pltpu.make_async_copy(w_b1, num_sh.at[dst_a.at[last]], sw1).wait()

    plsc.subcore_barrier()
    pltpu.sync_copy(num_sh.at[pl.ds(sid * rows_per, rows_per)],
                    out_hbm.at[cid, pl.ds(sid * rows_per, rows_per)])

  return sc_kernel(xlp, xr, src2, dst2, atab, zeros)


def kernel(x, edge_index, W1l, W1r, a1, b1, W2l, W2r, a2, b2):
  N, D = x.shape
  H = W1l.shape[1]
  E = edge_index.shape[1]
  C = 80
  NP = -(-N // 1024) * 1024

  nchunks = E // (NC * NS) // C
  src = edge_index[0].reshape(NC * NS * nchunks, C)
  dst = edge_index[1].reshape(NC * NS * nchunks, C)
  xp = jnp.pad(x, ((0, NP - N), (0, 0)))
  zeros = jnp.zeros((NP, W), jnp.float32)
  atab1 = jnp.broadcast_to(a1[:, None], (H, LANES)).astype(jnp.float32)
  atab2 = jnp.broadcast_to(a2[:, None], (H, LANES)).astype(jnp.float32)
  b1r = b1.reshape(1, H)
  b2r = b2.reshape(1, H)

  xlp1, xr1 = _tc_transform(xp, W1l, W1r)
  part1 = _sc_edge_pass(xlp1, xr1, src, dst, atab1, zeros, N=NP, E=E, H=H, C=C)
  xlp2, xr2 = _tc_combine_transform(part1, b1r, W2l, W2r)
  part2 = _sc_edge_pass(xlp2, xr2, src, dst, atab2, zeros, N=NP, E=E, H=H, C=C)
  return _tc_finalize(part2, b2r)[:N]


# speedup vs baseline: 1.6236x; 1.6236x over previous
import functools

import jax
import jax.numpy as jnp
from jax import lax
from jax.experimental import pallas as pl
from jax.experimental.pallas import tpu as pltpu
from jax.experimental.pallas import tpu_sc as plsc

NC = 2
NS = 16
LANES = 16
W = 33


def _tc_transform(feats, Wl, Wr):
  N, D = feats.shape
  H = Wl.shape[1]
  RB = 1024
  grid = (N // RB,)

  def body(f_ref, wl_ref, wr_ref, xlp_ref, xr_ref):
    f = f_ref[...]
    xl = jnp.dot(f, wl_ref[...], preferred_element_type=jnp.float32)
    xr = jnp.dot(f, wr_ref[...], preferred_element_type=jnp.float32)
    ones = jnp.ones((RB, 1), jnp.float32)
    zeros = jnp.zeros((RB, 1), jnp.float32)
    xlp_ref[...] = jnp.concatenate([xl, ones], axis=1)
    xr_ref[...] = jnp.concatenate([xr, zeros], axis=1)

  return pl.pallas_call(
      body,
      grid=grid,
      in_specs=[
          pl.BlockSpec((RB, D), lambda i: (i, 0)),
          pl.BlockSpec((D, H), lambda i: (0, 0)),
          pl.BlockSpec((D, H), lambda i: (0, 0)),
      ],
      out_specs=[
          pl.BlockSpec((RB, W), lambda i: (i, 0)),
          pl.BlockSpec((RB, W), lambda i: (i, 0)),
      ],
      out_shape=[
          jax.ShapeDtypeStruct((N, W), jnp.float32),
          jax.ShapeDtypeStruct((N, W), jnp.float32),
      ],
  )(feats, Wl, Wr)


def _tc_combine_transform(part, b, Wl, Wr):
  _, N, _ = part.shape
  H = Wl.shape[1]
  RB = 1024
  grid = (N // RB,)

  def body(p_ref, b_ref, wl_ref, wr_ref, xlp_ref, xr_ref):
    p = p_ref[0] + p_ref[1]
    den = jnp.maximum(p[:, H:H + 1], 1e-16)
    h = jnp.maximum(p[:, :H] / den + b_ref[...], 0.0)
    xl = jnp.dot(h, wl_ref[...], preferred_element_type=jnp.float32)
    xr = jnp.dot(h, wr_ref[...], preferred_element_type=jnp.float32)
    ones = jnp.ones((RB, 1), jnp.float32)
    zeros = jnp.zeros((RB, 1), jnp.float32)
    xlp_ref[...] = jnp.concatenate([xl, ones], axis=1)
    xr_ref[...] = jnp.concatenate([xr, zeros], axis=1)

  return pl.pallas_call(
      body,
      grid=grid,
      in_specs=[
          pl.BlockSpec((NC, RB, W), lambda i: (0, i, 0)),
          pl.BlockSpec((1, H), lambda i: (0, 0)),
          pl.BlockSpec((H, H), lambda i: (0, 0)),
          pl.BlockSpec((H, H), lambda i: (0, 0)),
      ],
      out_specs=[
          pl.BlockSpec((RB, W), lambda i: (i, 0)),
          pl.BlockSpec((RB, W), lambda i: (i, 0)),
      ],
      out_shape=[
          jax.ShapeDtypeStruct((N, W), jnp.float32),
          jax.ShapeDtypeStruct((N, W), jnp.float32),
      ],
  )(part, b, Wl, Wr)


def _tc_finalize(part, b):
  _, N, _ = part.shape
  H = W - 1
  RB = 1024
  grid = (N // RB,)

  def body(p_ref, b_ref, out_ref):
    p = p_ref[0] + p_ref[1]
    den = jnp.maximum(p[:, H:H + 1], 1e-16)
    out_ref[...] = p[:, :H] / den + b_ref[...]

  return pl.pallas_call(
      body,
      grid=grid,
      in_specs=[
          pl.BlockSpec((NC, RB, W), lambda i: (0, i, 0)),
          pl.BlockSpec((1, H), lambda i: (0, 0)),
      ],
      out_specs=pl.BlockSpec((RB, H), lambda i: (i, 0)),
      out_shape=jax.ShapeDtypeStruct((N, H), jnp.float32),
  )(part, b)


def _sc_edge_pass(xlp, xr, src2, dst2, atab, zeros, *, N, E, H, C):
  G = C // LANES
  per_worker = E // (NC * NS)
  nchunks = per_worker // C
  npairs = (nchunks - 1) // 2
  rows_per = N // NS

  mesh = plsc.VectorSubcoreMesh(core_axis_name="c", subcore_axis_name="s")

  @functools.partial(
      pl.kernel,
      out_type=jax.ShapeDtypeStruct((NC, N, W), jnp.float32),
      mesh=mesh,
      compiler_params=pltpu.CompilerParams(
          use_tc_tiling_on_sc=False, needs_layout_passes=False),
      scratch_types=[
          pltpu.VMEM((nchunks, C), jnp.int32),
          pltpu.VMEM((nchunks, C), jnp.int32),
          pltpu.VMEM((C, W), jnp.float32),
          pltpu.VMEM((C, W), jnp.float32),
          pltpu.VMEM((C, W), jnp.float32),
          pltpu.VMEM((C, W), jnp.float32),
          pltpu.VMEM((C, W), jnp.float32),
          pltpu.VMEM((C, W), jnp.float32),
          pltpu.VMEM((H, LANES), jnp.float32),
          pltpu.VMEM_SHARED((N, W), jnp.float32),
          pltpu.SemaphoreType.DMA,
          pltpu.SemaphoreType.DMA,
          pltpu.SemaphoreType.DMA,
          pltpu.SemaphoreType.DMA,
          pltpu.SemaphoreType.DMA,
          pltpu.SemaphoreType.DMA,
      ],
  )
  def sc_kernel(xlp_hbm, xr_hbm, src_hbm, dst_hbm, atab_hbm, zeros_hbm,
                out_hbm, src_a, dst_a, xlp_b0, xr_b0, xlp_b1, xr_b1, w_b0,
                w_b1, atab_v, num_sh, sl0, sr0, sl1, sr1, sw0, sw1):
    cid = lax.axis_index("c")
    sid = lax.axis_index("s")
    wid = sid * NC + cid

    pltpu.sync_copy(src_hbm.at[pl.ds(wid * nchunks, nchunks)], src_a)
    pltpu.sync_copy(dst_hbm.at[pl.ds(wid * nchunks, nchunks)], dst_a)
    pltpu.sync_copy(atab_hbm, atab_v)
    pltpu.async_copy(xlp_hbm.at[src_a.at[0]], xlp_b0, sl0)
    pltpu.async_copy(xr_hbm.at[dst_a.at[0]], xr_b0, sr0)
    pltpu.sync_copy(zeros_hbm.at[pl.ds(sid * rows_per, rows_per)],
                    num_sh.at[pl.ds(sid * rows_per, rows_per)])
    plsc.subcore_barrier()

    iota = lax.iota(jnp.int32, LANES)
    rows_g = [iota + g * LANES for g in range(G)]
    col_den = jnp.full((LANES,), H, jnp.int32)

    def compute_chunk(xlp_b, xr_b, w_b, sw, j, wait_w):
      UN = 4

      def e_body(hh, accs):
        h0 = hh * UN
        out = list(accs)
        for dh in range(UN):
          hv = jnp.full((LANES,), h0 + dh, jnp.int32)
          a_h = plsc.load_gather(atab_v, [hv, iota])
          for g in range(G):
            tl = plsc.load_gather(xlp_b, [rows_g[g], hv])
            tr = plsc.load_gather(xr_b, [rows_g[g], hv])
            t = tl + tr
            t = jnp.maximum(t, 0.2 * t)
            out[g] = out[g] + a_h * t
        return tuple(out)

      zero16 = jnp.zeros((LANES,), jnp.float32)
      accs = lax.fori_loop(0, H // UN, e_body, tuple(zero16 for _ in range(G)))
      ees = [jnp.exp(jnp.minimum(a, 60.0)) for a in accs]

      @pl.when(wait_w)
      def _():
        pltpu.make_async_copy(w_b, num_sh.at[dst_a.at[j]], sw).wait()

      for g in range(G):
        plsc.store_scatter(w_b, [rows_g[g], col_den], ees[g])

      def w_body(hh, carry2):
        h0 = hh * UN
        for dh in range(UN):
          hv = jnp.full((LANES,), h0 + dh, jnp.int32)
          for g in range(G):
            tl = plsc.load_gather(xlp_b, [rows_g[g], hv])
            plsc.store_scatter(w_b, [rows_g[g], hv], ees[g] * tl)
        return carry2

      lax.fori_loop(0, H // UN, w_body, 0)
      pltpu.async_copy(w_b, num_sh.at[dst_a.at[j]], sw, add=True)

    def pair_body(i, carry):
      j0 = 2 * i
      pltpu.make_async_copy(xlp_hbm.at[src_a.at[j0]], xlp_b0, sl0).wait()
      pltpu.make_async_copy(xr_hbm.at[dst_a.at[j0]], xr_b0, sr0).wait()
      pltpu.async_copy(xlp_hbm.at[src_a.at[j0 + 1]], xlp_b1, sl1)
      pltpu.async_copy(xr_hbm.at[dst_a.at[j0 + 1]], xr_b1, sr1)
      compute_chunk(xlp_b0, xr_b0, w_b0, sw0, j0, i > 0)
      pltpu.make_async_copy(xlp_hbm.at[src_a.at[j0 + 1]], xlp_b1, sl1).wait()
      pltpu.make_async_copy(xr_hbm.at[dst_a.at[j0 + 1]], xr_b1, sr1).wait()
      pltpu.async_copy(xlp_hbm.at[src_a.at[j0 + 2]], xlp_b0, sl0)
      pltpu.async_copy(xr_hbm.at[dst_a.at[j0 + 2]], xr_b0, sr0)
      compute_chunk(xlp_b1, xr_b1, w_b1, sw1, j0 + 1, i > 0)
      return carry

    lax.fori_loop(0, npairs, pair_body, 0)

    last = nchunks - 1
    pltpu.make_async_copy(xlp_hbm.at[src_a.at[last]], xlp_b0, sl0).wait()
    pltpu.make_async_copy(xr_hbm.at[dst_a.at[last]], xr_b0, sr0).wait()
    compute_chunk(xlp_b0, xr_b0, w_b0, sw0, last, jnp.bool_(True))
    pltpu.make_async_copy(w_b0, num_sh.at[dst_a.at[last]], sw0).wait()
    pltpu.make_async_copy(w_b1, num_sh.at[dst_a.at[last]], sw1).wait()

    plsc.subcore_barrier()
    pltpu.sync_copy(num_sh.at[pl.ds(sid * rows_per, rows_per)],
                    out_hbm.at[cid, pl.ds(sid * rows_per, rows_per)])

  return sc_kernel(xlp, xr, src2, dst2, atab, zeros)


def kernel(x, edge_index, W1l, W1r, a1, b1, W2l, W2r, a2, b2):
  N, D = x.shape
  H = W1l.shape[1]
  E = edge_index.shape[1]
  C = 80
  NP = -(-N // 1024) * 1024

  nchunks = E // (NC * NS) // C
  src = edge_index[0].reshape(NC * NS * nchunks, C)
  dst = edge_index[1].reshape(NC * NS * nchunks, C)
  xp = jnp.pad(x, ((0, NP - N), (0, 0)))
  zeros = jnp.zeros((NP, W), jnp.float32)
  atab1 = jnp.broadcast_to(a1[:, None], (H, LANES)).astype(jnp.float32)
  atab2 = jnp.broadcast_to(a2[:, None], (H, LANES)).astype(jnp.float32)
  b1r = b1.reshape(1, H)
  b2r = b2.reshape(1, H)

  xlp1, xr1 = _tc_transform(xp, W1l, W1r)
  part1 = _sc_edge_pass(xlp1, xr1, src, dst, atab1, zeros, N=NP, E=E, H=H, C=C)
  xlp2, xr2 = _tc_combine_transform(part1, b1r, W2l, W2r)
  part2 = _sc_edge_pass(xlp2, xr2, src, dst, atab2, zeros, N=NP, E=E, H=H, C=C)
  return _tc_finalize(part2, b2r)[:N]
